# Initial kernel scaffold; baseline (speedup 1.0000x reference)
import jax
import jax.numpy as jnp
from jax.experimental import pallas as pl


def kernel(x, tok_table):
    raise NotImplementedError("write your pallas kernel here")


# speedup vs baseline: 1.4013x; 1.4013x over previous
import functools

import numpy as np
import jax
import jax.numpy as jnp
from jax import lax
from jax.experimental import pallas as pl
from jax.experimental.pallas import tpu as pltpu
from jax.experimental.pallas import tpu_sc as plsc

_LANES = 16


@functools.lru_cache(maxsize=None)
def _pos_table(seq_len: int, d_model: int):
    pos = np.arange(seq_len, dtype=np.float32)[:, None]
    i = np.arange(0, d_model, 2, dtype=np.float32)
    div = np.power(np.float32(10000.0), i / np.float32(d_model)).astype(np.float32)
    pe = np.zeros((seq_len, d_model), dtype=np.float32)
    pe[:, 0::2] = np.sin(pos / div)
    pe[:, 1::2] = np.cos(pos / div)
    return jnp.asarray(pe)


@functools.lru_cache(maxsize=None)
def _make_embed(N: int, S: int, D: int, C: int):
    info = plsc.get_sparse_core_info()
    nw = info.num_cores * info.num_subcores
    b_per_w = N // nw
    n_chunks = b_per_w // C
    assert b_per_w % C == 0 and D % _LANES == 0 and S % b_per_w == 0

    mesh = plsc.VectorSubcoreMesh(core_axis_name="c", subcore_axis_name="s")

    @functools.partial(
        pl.kernel,
        mesh=mesh,
        out_type=jax.ShapeDtypeStruct((N, D), jnp.float32),
        scratch_types=[
            pltpu.VMEM((C,), jnp.int32),
            pltpu.VMEM((C, D), jnp.float32),
            pltpu.VMEM((C, D), jnp.float32),
            pltpu.SemaphoreType.DMA,
        ],
    )
    def k(table_hbm, idx_hbm, pos_hbm, out_hbm, idx_v, tok_v, pos_v, sem):
        wid = lax.axis_index("s") * info.num_cores + lax.axis_index("c")
        base = wid * b_per_w
        pos_base = lax.rem(base, S)
        for c in range(n_chunks):
            row0 = base + c * C
            pltpu.sync_copy(idx_hbm.at[pl.ds(row0, C)], idx_v)
            gather = pltpu.async_copy(table_hbm.at[idx_v], tok_v, sem)
            pltpu.sync_copy(pos_hbm.at[pl.ds(pos_base + c * C, C)], pos_v)
            gather.wait()

            def body(i, carry):
                for j in range(D // _LANES):
                    sl = pl.ds(j * _LANES, _LANES)
                    tok_v[i, sl] = tok_v[i, sl] + pos_v[i, sl]
                return carry

            lax.fori_loop(0, C, body, 0)
            pltpu.sync_copy(tok_v, out_hbm.at[pl.ds(row0, C)])

    return k


def kernel(x, tok_table):
    B, S = x.shape
    V, D = tok_table.shape
    N = B * S
    idx = x.reshape(N).astype(jnp.int32)
    pos = _pos_table(S, D)
    out = _make_embed(N, S, D, 64)(tok_table, idx, pos)
    return out.reshape(B, S, D)


# speedup vs baseline: 2.1465x; 1.5317x over previous
import functools

import numpy as np
import jax
import jax.numpy as jnp
from jax import lax
from jax.experimental import pallas as pl
from jax.experimental.pallas import tpu as pltpu
from jax.experimental.pallas import tpu_sc as plsc

_LANES = 16


@functools.lru_cache(maxsize=None)
def _pos_table(seq_len: int, d_model: int):
    pos = np.arange(seq_len, dtype=np.float32)[:, None]
    i = np.arange(0, d_model, 2, dtype=np.float32)
    div = np.power(np.float32(10000.0), i / np.float32(d_model)).astype(np.float32)
    pe = np.zeros((seq_len, d_model), dtype=np.float32)
    pe[:, 0::2] = np.sin(pos / div)
    pe[:, 1::2] = np.cos(pos / div)
    return jnp.asarray(pe)


@functools.lru_cache(maxsize=None)
def _make_embed(B: int, S: int, D: int, P: int):
    info = plsc.get_sparse_core_info()
    nc = info.num_cores
    nw = nc * info.num_subcores
    pos_per_w = S // nw
    n_p = pos_per_w // P
    T = n_p * B
    N = B * S
    assert pos_per_w % P == 0 and D % _LANES == 0

    mesh = plsc.VectorSubcoreMesh(core_axis_name="c", subcore_axis_name="s")

    @functools.partial(
        pl.kernel,
        mesh=mesh,
        out_type=jax.ShapeDtypeStruct((N, D), jnp.float32),
        scratch_types=[
            pltpu.VMEM((T * P,), jnp.int32),
            pltpu.VMEM((P, D), jnp.float32),
            pltpu.VMEM((P, D), jnp.float32),
            pltpu.VMEM((P, D), jnp.float32),
            pltpu.VMEM((P, D), jnp.float32),
            pltpu.VMEM((P, D), jnp.float32),
            pltpu.SemaphoreType.DMA,
            pltpu.SemaphoreType.DMA,
            pltpu.SemaphoreType.DMA,
            pltpu.SemaphoreType.DMA,
            pltpu.SemaphoreType.DMA,
            pltpu.SemaphoreType.DMA,
            pltpu.SemaphoreType.DMA,
        ],
    )
    def k(table_hbm, idx_hbm, pos_hbm, out_hbm,
          idxall, tk0, tk1, tk2, q0, q1,
          g0, g1, g2, o0, o1, o2, pp):
        wid = lax.axis_index("s") * nc + lax.axis_index("c")
        toks = [tk0, tk1, tk2]
        poss = [q0, q1]
        gsem = [g0, g1, g2]
        osem = [o0, o1, o2]
        pos0 = wid * pos_per_w

        pltpu.sync_copy(idx_hbm.at[pl.ds(wid * (T * P), T * P)], idxall)
        pltpu.sync_copy(pos_hbm.at[pl.ds(pos0, P)], q0)
        pos_pf = None
        if n_p > 1:
            pos_pf = pltpu.async_copy(pos_hbm.at[pl.ds(pos0 + P, P)], q1, pp)

        gathers = [None] * T
        stores = [None] * T
        gathers[0] = pltpu.async_copy(
            table_hbm.at[idxall.at[pl.ds(0, P)]], toks[0], gsem[0])

        for t in range(T):
            p, b = divmod(t, B)
            s = t % 3
            if t + 1 < T:
                if t - 2 >= 0:
                    stores[t - 2].wait()
                s1 = (t + 1) % 3
                gathers[t + 1] = pltpu.async_copy(
                    table_hbm.at[idxall.at[pl.ds((t + 1) * P, P)]],
                    toks[s1], gsem[s1])
            if b == 0 and p > 0:
                pos_pf.wait()
                if p + 1 < n_p:
                    pos_pf = pltpu.async_copy(
                        pos_hbm.at[pl.ds(pos0 + (p + 1) * P, P)],
                        poss[(p + 1) % 2], pp)
            gathers[t].wait()
            tk = toks[s]
            pq = poss[p % 2]

            def body(i, carry, tk=tk, pq=pq):
                for j in range(D // _LANES):
                    sl = pl.ds(j * _LANES, _LANES)
                    tk[i, sl] = tk[i, sl] + pq[i, sl]
                return carry

            lax.fori_loop(0, P, body, 0)
            row0 = b * S + pos0 + p * P
            stores[t] = pltpu.async_copy(
                tk, out_hbm.at[pl.ds(row0, P)], osem[s])

        for t in range(max(0, T - 3), T):
            stores[t].wait()

    return k


def kernel(x, tok_table):
    B, S = x.shape
    V, D = tok_table.shape
    P = 32
    nw = 32
    pos_per_w = S // nw
    n_p = pos_per_w // P
    idx = (x.astype(jnp.int32)
             .reshape(B, nw, n_p, P)
             .transpose(1, 2, 0, 3)
             .reshape(nw * n_p * B * P))
    pos = _pos_table(S, D)
    out = _make_embed(B, S, D, P)(tok_table, idx, pos)
    return out.reshape(B, S, D)
